# baseline (speedup 1.0000x reference)
import jax
import jax.numpy as jnp
import numpy as np
from jax.experimental import pallas as pl

B, N, N1, N2, CGEO = 16, 4096, 512, 128, 256
R1, R2, KFP, MAXN1, MAXN2 = 0.2, 0.4, 3, 32, 64


def _bn_relu(x, g, b):
    xs = x.reshape(-1, x.shape[-1])
    mean = xs.mean(axis=0)
    var = xs.var(axis=0)
    y = (x - mean) / jnp.sqrt(var + 1e-5) * g + b
    return jnp.maximum(y, 0.0)


def _apply_mlp(x, layers):
    for (W, bb, gm, bt) in layers:
        x = _bn_relu(x @ W + bb, gm, bt)
    return x


def _batched_gather(x, idx):
    return jax.vmap(lambda xb, ib: xb[ib])(x, idx)


def _fps(pos, npoint):
    pos = jax.lax.stop_gradient(pos)
    Bb, Nn, _ = pos.shape

    def body(i, state):
        idxs, dists, far = state
        idxs = idxs.at[:, i].set(far)
        centroid = jnp.take_along_axis(pos, far[:, None, None].astype(jnp.int32), axis=1)
        d = jnp.sum((pos - centroid) ** 2, axis=-1)
        dists = jnp.minimum(dists, d)
        far = jnp.argmax(dists, axis=-1).astype(jnp.int32)
        return idxs, dists, far

    idxs = jnp.zeros((Bb, npoint), jnp.int32)
    dists = jnp.full((Bb, Nn), 1e10, jnp.float32)
    far = jnp.zeros((Bb,), jnp.int32)
    idxs, _, _ = jax.lax.fori_loop(0, npoint, body, (idxs, dists, far))
    return idxs


def _ball_query(pos_src, pos_q, r, nsample):
    ps = jax.lax.stop_gradient(pos_src)
    pq = jax.lax.stop_gradient(pos_q)
    d2 = jnp.sum((pq[:, :, None, :] - ps[:, None, :, :]) ** 2, axis=-1)
    Ns = pos_src.shape[1]
    ar = jnp.arange(Ns, dtype=jnp.int32)
    gidx = jnp.where(d2 <= r * r, ar[None, None, :], Ns)
    gidx = jnp.sort(gidx, axis=-1)[:, :, :nsample]
    first = gidx[:, :, :1]
    gidx = jnp.where(gidx == Ns, first, gidx)
    return gidx


def _knn_interpolate(x, pos_src, pos_tgt, k):
    d2 = jnp.sum((pos_tgt[:, :, None, :] - pos_src[:, None, :, :]) ** 2, axis=-1)
    negd, idx = jax.lax.top_k(-d2, k)
    w = 1.0 / jnp.maximum(-negd, 1e-16)
    xk = _batched_gather(x, idx)
    return jnp.sum(w[..., None] * xk, axis=2) / jnp.sum(w, axis=-1, keepdims=True)


def _identity_kernel(x_ref, o_ref):
    o_ref[...] = x_ref[...]


def _pallas_identity(x):
    return pl.pallas_call(
        _identity_kernel,
        grid=(x.shape[0],),
        in_specs=[pl.BlockSpec((1,) + x.shape[1:], lambda b: (b, 0, 0))],
        out_specs=pl.BlockSpec((1,) + x.shape[1:], lambda b: (b, 0, 0)),
        out_shape=jax.ShapeDtypeStruct(x.shape, x.dtype),
    )(x)


def kernel(pts, params):
    x0 = pts
    idx1 = _fps(pts, N1)
    pos1 = _batched_gather(pts, idx1)
    g1 = _ball_query(pts, pos1, R1, MAXN1)
    xj = _batched_gather(x0, g1)
    pj = _batched_gather(pts, g1)
    h = _apply_mlp(jnp.concatenate([xj, pj - pos1[:, :, None, :]], axis=-1), params['sa1_local'])
    x1 = _apply_mlp(jnp.max(h, axis=2), params['sa1_global'])
    idx2 = _fps(pos1, N2)
    pos2 = _batched_gather(pos1, idx2)
    g2 = _ball_query(pos1, pos2, R2, MAXN2)
    xj = _batched_gather(x1, g2)
    pj = _batched_gather(pos1, g2)
    h = _apply_mlp(jnp.concatenate([xj, pj - pos2[:, :, None, :]], axis=-1), params['sa2_local'])
    x2 = _apply_mlp(jnp.max(h, axis=2), params['sa2_global'])
    g = _apply_mlp(jnp.max(x2, axis=1), params['glob'])
    x1_up = _knn_interpolate(x2, pos2, pos1, KFP)
    x1_fp = _apply_mlp(jnp.concatenate([x1_up, x1], axis=-1), params['fp1'])
    x0_up = _knn_interpolate(x1_fp, pos1, pts, KFP)
    F = _apply_mlp(jnp.concatenate([x0_up, x0], axis=-1), params['fp0'])
    return (_pallas_identity(F), g)


# speedup vs baseline: 4.7798x; 4.7798x over previous
import jax
import jax.numpy as jnp
from jax import lax
from jax.experimental import pallas as pl
from jax.experimental.pallas import tpu as pltpu

B, N, N1, N2, CGEO = 16, 4096, 512, 128, 256
R1, R2, KFP, MAXN1, MAXN2 = 0.2, 0.4, 3, 32, 64
_EPS = 1e-5
_HI = lax.Precision.HIGHEST


def _pack(*rows):
    z = jnp.zeros_like(rows[0])
    return jnp.stack(list(rows) + [z] * (8 - len(rows)))


def _fps_call(posT, npoint):
    _, Bb, Nn = posT.shape

    def kern(pt_ref, oq_ref, dist_ref):
        px, py, pz = pt_ref[0], pt_ref[1], pt_ref[2]
        lane = lax.broadcasted_iota(jnp.int32, (Bb, Nn), 1).astype(jnp.float32)
        olane = lax.broadcasted_iota(jnp.int32, (Bb, npoint), 1).astype(
            jnp.float32)
        dist_ref[...] = jnp.full((Bb, Nn), 1e10, jnp.float32)
        oq_ref[...] = jnp.zeros((3, Bb, npoint), jnp.float32)

        def body(i, far):
            oh = (lane == far).astype(jnp.float32)
            cx = jnp.sum(px * oh, axis=1, keepdims=True)
            cy = jnp.sum(py * oh, axis=1, keepdims=True)
            cz = jnp.sum(pz * oh, axis=1, keepdims=True)
            ohi = (olane == i.astype(jnp.float32)).astype(jnp.float32)
            oq_ref[0] += cx * ohi
            oq_ref[1] += cy * ohi
            oq_ref[2] += cz * ohi
            d = (px - cx) ** 2 + (py - cy) ** 2 + (pz - cz) ** 2
            nd = jnp.minimum(dist_ref[...], d)
            dist_ref[...] = nd
            m = jnp.max(nd, axis=1, keepdims=True)
            return jnp.min(jnp.where(nd == m, lane, float(Nn)), axis=1,
                           keepdims=True)

        lax.fori_loop(0, npoint, body, jnp.zeros((Bb, 1), jnp.float32))

    return pl.pallas_call(
        kern,
        out_shape=jax.ShapeDtypeStruct((3, Bb, npoint), jnp.float32),
        scratch_shapes=[pltpu.VMEM((Bb, Nn), jnp.float32)],
    )(posT)


def _ball_call(qpos, srcB, r, ns, qblk):
    Bb, Q, _ = qpos.shape
    Nn = srcB.shape[2]

    def kern(q_ref, s_ref, o_ref):
        b = pl.program_id(0)
        q = q_ref[0]
        s3 = s_ref[0]
        d2 = jnp.zeros((qblk, Nn), jnp.float32)
        for c in range(3):
            d2 = d2 + (q[:, c:c + 1] - s3[c:c + 1]) ** 2
        csum = (d2 <= r * r).astype(jnp.float32)
        sh = 1
        while sh < Nn:
            csum = csum + jnp.concatenate(
                [jnp.zeros((qblk, sh), jnp.float32), csum[:, :Nn - sh]], axis=1)
            sh *= 2
        cols = [jnp.sum((csum <= float(s)).astype(jnp.float32), axis=1,
                        keepdims=True) for s in range(ns)]
        g = jnp.concatenate(cols, axis=1)
        g = jnp.where(g == float(Nn), g[:, 0:1], g)
        o_ref[0] = g.astype(jnp.int32) + b * Nn

    return pl.pallas_call(
        kern, grid=(Bb, Q // qblk),
        in_specs=[pl.BlockSpec((1, qblk, 3), lambda b, q: (b, q, 0)),
                  pl.BlockSpec((1, 3, Nn), lambda b, q: (b, 0, 0))],
        out_specs=pl.BlockSpec((1, qblk, ns), lambda b, q: (b, q, 0)),
        out_shape=jax.ShapeDtypeStruct((Bb, Q, ns), jnp.int32),
    )(qpos, srcB)


def _sa1_first_call(pts, pos1, gflat, W1, P1, qblk, nblk):
    S = MAXN1
    rows = qblk * S
    nn = N // nblk
    Cout = W1.shape[1]

    def kern(p_ref, q_ref, g_ref, w_ref, bp_ref, z_ref, pj_ref):
        b, qi, ni = pl.program_id(0), pl.program_id(1), pl.program_id(2)
        gl = (g_ref[0] - b * N).astype(jnp.float32)
        jio = (lax.broadcasted_iota(jnp.int32, (nblk, 1), 0).astype(
            jnp.float32) + (ni * nblk).astype(jnp.float32))
        oh = (jio == gl).astype(jnp.float32)
        part = lax.dot_general(oh, p_ref[0], (((0,), (0,)), ((), ())),
                               preferred_element_type=jnp.float32,
                               precision=_HI)

        @pl.when(ni == 0)
        def _():
            pj_ref[...] = jnp.zeros((rows, 3), jnp.float32)

        pj_ref[...] += part

        @pl.when(ni == nn - 1)
        def _():
            pj = pj_ref[...]
            pq = jnp.broadcast_to(q_ref[0][:, None, :],
                                  (qblk, S, 3)).reshape(rows, 3)
            feat = jnp.concatenate([pj, pj - pq], axis=1)
            z_ref[0] = jnp.dot(
                feat, w_ref[...],
                preferred_element_type=jnp.float32) + bp_ref[0:1]

    return pl.pallas_call(
        kern, grid=(B, N1 // qblk, nn),
        in_specs=[
            pl.BlockSpec((1, nblk, 3), lambda b, q, n: (b, n, 0)),
            pl.BlockSpec((1, qblk, 3), lambda b, q, n: (b, q, 0)),
            pl.BlockSpec((1, 1, rows), lambda b, q, n: (b, 0, q)),
            pl.BlockSpec((6, Cout), lambda b, q, n: (0, 0)),
            pl.BlockSpec((8, Cout), lambda b, q, n: (0, 0)),
        ],
        out_specs=pl.BlockSpec((1, rows, Cout), lambda b, q, n: (b, q, 0)),
        out_shape=jax.ShapeDtypeStruct((B, N1 * S, Cout), jnp.float32),
        scratch_shapes=[pltpu.VMEM((rows, 3), jnp.float32)],
    )(pts, pos1, gflat, W1, P1)


def _sa2_first_call(x1, pos1, pos2, gflat, V1, Q1, qblk):
    S = MAXN2
    rows = qblk * S
    C = x1.shape[2]
    Cout = V1.shape[1]

    def kern(x_ref, p_ref, q_ref, g_ref, w_ref, bp_ref, z_ref):
        b = pl.program_id(0)
        gl = (g_ref[0] - b * N1).astype(jnp.float32)
        jio = lax.broadcasted_iota(jnp.int32, (N1, 1), 0).astype(jnp.float32)
        oh = (jio == gl).astype(jnp.float32)
        xj = lax.dot_general(oh, x_ref[0], (((0,), (0,)), ((), ())),
                             preferred_element_type=jnp.float32,
                             precision=_HI)
        pj = lax.dot_general(oh, p_ref[0], (((0,), (0,)), ((), ())),
                             preferred_element_type=jnp.float32,
                             precision=_HI)
        pq = jnp.broadcast_to(q_ref[0][:, None, :],
                              (qblk, S, 3)).reshape(rows, 3)
        feat = jnp.concatenate([xj, pj - pq], axis=1)
        z_ref[0] = jnp.dot(feat, w_ref[...],
                           preferred_element_type=jnp.float32) + bp_ref[0:1]

    return pl.pallas_call(
        kern, grid=(B, N2 // qblk),
        in_specs=[
            pl.BlockSpec((1, N1, C), lambda b, q: (b, 0, 0)),
            pl.BlockSpec((1, N1, 3), lambda b, q: (b, 0, 0)),
            pl.BlockSpec((1, qblk, 3), lambda b, q: (b, q, 0)),
            pl.BlockSpec((1, 1, rows), lambda b, q: (b, 0, q)),
            pl.BlockSpec((C + 3, Cout), lambda b, q: (0, 0)),
            pl.BlockSpec((8, Cout), lambda b, q: (0, 0)),
        ],
        out_specs=pl.BlockSpec((1, rows, Cout), lambda b, q: (b, q, 0)),
        out_shape=jax.ShapeDtypeStruct((B, N2 * S, Cout), jnp.float32),
    )(x1, pos1, pos2, gflat, V1, Q1)


def _bn_layer_call(X, MV, Pin, W, Pout, pool, blk, prec=None):
    R, Cin = X.shape
    gridn = R // blk
    rows_out = blk // pool if pool else blk
    Cout = W.shape[1] if W is not None else Cin

    def kern(x_ref, mv_ref, pi_ref, *rest):
        mv = mv_ref[...]
        y = jnp.maximum(
            (x_ref[...] - mv[0:1]) / jnp.sqrt(mv[1:2] + _EPS) * pi_ref[1:2]
            + pi_ref[2:3], 0.0)
        if pool:
            y = jnp.max(y.reshape(rows_out, pool, Cin), axis=1)
        if W is None:
            rest[0][...] = y
            return
        w_ref, po_ref, z_ref = rest
        z_ref[...] = jnp.dot(y, w_ref[...], preferred_element_type=jnp.float32,
                             precision=prec) + po_ref[0:1]

    in_specs = [pl.BlockSpec((blk, Cin), lambda i: (i, 0)),
                pl.BlockSpec((8, Cin), lambda i: (0, 0)),
                pl.BlockSpec((8, Cin), lambda i: (0, 0))]
    args = [X, MV, Pin]
    if W is None:
        return pl.pallas_call(
            kern, grid=(gridn,), in_specs=in_specs,
            out_specs=pl.BlockSpec((rows_out, Cin), lambda i: (i, 0)),
            out_shape=jax.ShapeDtypeStruct((R // (pool or 1), Cin),
                                           jnp.float32),
        )(*args)
    in_specs += [pl.BlockSpec((Cin, Cout), lambda i: (0, 0)),
                 pl.BlockSpec((8, Cout), lambda i: (0, 0))]
    args += [W, Pout]
    return pl.pallas_call(
        kern, grid=(gridn,), in_specs=in_specs,
        out_specs=pl.BlockSpec((rows_out, Cout), lambda i: (i, 0)),
        out_shape=jax.ShapeDtypeStruct((R // (pool or 1), Cout), jnp.float32),
    )(*args)


def _concat_layer_call(X1, X2, W, Pout, blk, prec=None):
    R, C1 = X1.shape
    C2 = X2.shape[1]
    Cout = W.shape[1]
    gridn = R // blk

    def kern(x1_ref, x2_ref, w_ref, po_ref, z_ref):
        feat = jnp.concatenate([x1_ref[...], x2_ref[...]], axis=1)
        z_ref[...] = jnp.dot(feat, w_ref[...], preferred_element_type=jnp.float32,
                             precision=prec) + po_ref[0:1]

    return pl.pallas_call(
        kern, grid=(gridn,),
        in_specs=[pl.BlockSpec((blk, C1), lambda i: (i, 0)),
                  pl.BlockSpec((blk, C2), lambda i: (i, 0)),
                  pl.BlockSpec((C1 + C2, Cout), lambda i: (0, 0)),
                  pl.BlockSpec((8, Cout), lambda i: (0, 0))],
        out_specs=pl.BlockSpec((blk, Cout), lambda i: (i, 0)),
        out_shape=jax.ShapeDtypeStruct((R, Cout), jnp.float32),
    )(X1, X2, W, Pout)


def _glob_call(x2, W1, P1, W2, P2):
    C2 = W2.shape[1]

    def bn(z, p):
        mu = jnp.mean(z, axis=0, keepdims=True)
        var = jnp.mean((z - mu) ** 2, axis=0, keepdims=True)
        return jnp.maximum((z - mu) / jnp.sqrt(var + _EPS) * p[1:2] + p[2:3],
                           0.0)

    def kern(x_ref, w1_ref, p1_ref, w2_ref, p2_ref, o_ref):
        gm = jnp.max(x_ref[...], axis=1)
        z = jnp.dot(gm, w1_ref[...], preferred_element_type=jnp.float32,
                    precision=_HI) + p1_ref[0:1]
        y = bn(z, p1_ref[...])
        z2 = jnp.dot(y, w2_ref[...], preferred_element_type=jnp.float32,
                     precision=_HI) + p2_ref[0:1]
        o_ref[...] = bn(z2, p2_ref[...])

    return pl.pallas_call(
        kern,
        out_shape=jax.ShapeDtypeStruct((B, C2), jnp.float32),
    )(x2, W1, P1, W2, P2)


def _knn_call(qpos, srcB, xsrc, qblk):
    Bb, Q, _ = qpos.shape
    S = srcB.shape[2]
    C = xsrc.shape[2]

    def kern(q_ref, s_ref, x_ref, o_ref):
        q = q_ref[0]
        s3 = s_ref[0]
        d2 = jnp.zeros((qblk, S), jnp.float32)
        for c in range(3):
            d2 = d2 + (q[:, c:c + 1] - s3[c:c + 1]) ** 2
        lane = lax.broadcasted_iota(jnp.int32, (qblk, S), 1).astype(
            jnp.float32)
        num = jnp.zeros((qblk, C), jnp.float32)
        den = jnp.zeros((qblk, 1), jnp.float32)
        for _ in range(KFP):
            mn = jnp.min(d2, axis=1, keepdims=True)
            fi = jnp.min(jnp.where(d2 == mn, lane, float(S)), axis=1,
                         keepdims=True)
            oh = (lane == fi).astype(jnp.float32)
            xk = lax.dot_general(oh, x_ref[0], (((1,), (0,)), ((), ())),
                                 preferred_element_type=jnp.float32,
                                 precision=_HI)
            w = 1.0 / jnp.maximum(mn, 1e-16)
            num = num + w * xk
            den = den + w
            d2 = jnp.where(oh > 0.0, 1e30, d2)
        o_ref[0] = num / den

    return pl.pallas_call(
        kern, grid=(Bb, Q // qblk),
        in_specs=[pl.BlockSpec((1, qblk, 3), lambda b, q: (b, q, 0)),
                  pl.BlockSpec((1, 3, S), lambda b, q: (b, 0, 0)),
                  pl.BlockSpec((1, S, C), lambda b, q: (b, 0, 0))],
        out_specs=pl.BlockSpec((1, qblk, C), lambda b, q: (b, q, 0)),
        out_shape=jax.ShapeDtypeStruct((Bb, Q, C), jnp.float32),
    )(qpos, srcB, xsrc)


def kernel(pts, params):
    ptsT = jnp.transpose(pts, (2, 0, 1))
    ptsB = jnp.transpose(pts, (0, 2, 1))
    pos1T = _fps_call(ptsT, N1)
    pos1 = jnp.transpose(pos1T, (1, 2, 0))
    pos1B = jnp.transpose(pos1T, (1, 0, 2))
    pos2T = _fps_call(pos1T, N2)
    pos2 = jnp.transpose(pos2T, (1, 2, 0))
    pos2B = jnp.transpose(pos2T, (1, 0, 2))
    g1 = _ball_call(pos1, ptsB, R1, MAXN1, qblk=128)
    g2 = _ball_call(pos2, pos1B, R2, MAXN2, qblk=128)

    def mv(z):
        return _pack(jnp.mean(z, axis=0), jnp.var(z, axis=0))

    (W1, b1, g1m, t1), (W2, b2, g2m, t2), (W3, b3, g3m, t3) = \
        params['sa1_local']
    P1, P2, P3 = _pack(b1, g1m, t1), _pack(b2, g2m, t2), _pack(b3, g3m, t3)
    (Wg, bg, gg, tg) = params['sa1_global'][0]
    Pg = _pack(bg, gg, tg)
    n1 = B * N1 * MAXN1
    Z1 = _sa1_first_call(pts, pos1, g1.reshape(B, 1, N1 * MAXN1),
                         W1, P1, qblk=128, nblk=512).reshape(n1, 64)
    Z2 = _bn_layer_call(Z1, mv(Z1), P1, W2, P2, None, 4096)
    Z3 = _bn_layer_call(Z2, mv(Z2), P2, W3, P3, None, 4096)
    Z4 = _bn_layer_call(Z3, mv(Z3), P3, Wg, Pg, MAXN1, 4096)
    x1 = _bn_layer_call(Z4, mv(Z4), Pg, None, None, None, 4096)
    x1r = x1.reshape(B, N1, 256)

    (V1, c1, h1, u1), (V2, c2, h2, u2), (V3, c3, h3, u3) = \
        params['sa2_local']
    Q1, Q2, Q3 = _pack(c1, h1, u1), _pack(c2, h2, u2), _pack(c3, h3, u3)
    (Vg, cg, hg, ug) = params['sa2_global'][0]
    Qg = _pack(cg, hg, ug)
    n2 = B * N2 * MAXN2
    Z5 = _sa2_first_call(x1r, pos1, pos2, g2.reshape(B, 1, N2 * MAXN2),
                         V1, Q1, qblk=64).reshape(n2, 128)
    Z6 = _bn_layer_call(Z5, mv(Z5), Q1, V2, Q2, None, 4096)
    Z7 = _bn_layer_call(Z6, mv(Z6), Q2, V3, Q3, None, 4096)
    Z8 = _bn_layer_call(Z7, mv(Z7), Q3, Vg, Qg, MAXN2, 4096)
    x2 = _bn_layer_call(Z8, mv(Z8), Qg, None, None, None, 2048)
    x2r = x2.reshape(B, N2, 256)

    (Ga, ba, ga, ta), (Gb, bb2, gb2, tb2) = params['glob']
    gout = _glob_call(x2r, Ga, _pack(ba, ga, ta), Gb, _pack(bb2, gb2, tb2))

    (F1, fb1, fg1, ft1), (F2, fb2, fg2, ft2) = params['fp1']
    Pf1, Pf2 = _pack(fb1, fg1, ft1), _pack(fb2, fg2, ft2)
    x1_up = _knn_call(pos1, pos2B, x2r, qblk=512)
    Z9 = _concat_layer_call(x1_up.reshape(B * N1, 256), x1, F1, Pf1,
                            4096, prec=_HI)
    Z10 = _bn_layer_call(Z9, mv(Z9), Pf1, F2, Pf2, None, 4096, prec=_HI)
    x1_fp = _bn_layer_call(Z10, mv(Z10), Pf2, None, None, None, 4096)

    (H1, hb1, hg1, ht1), (H2, hb2, hg2, ht2) = params['fp0']
    Ph1, Ph2 = _pack(hb1, hg1, ht1), _pack(hb2, hg2, ht2)
    x0_up = _knn_call(pts, pos1B, x1_fp.reshape(B, N1, 256), qblk=512)
    Z11 = _concat_layer_call(x0_up.reshape(B * N, 256),
                             pts.reshape(B * N, 3), H1, Ph1, 4096, prec=_HI)
    Z12 = _bn_layer_call(Z11, mv(Z11), Ph1, H2, Ph2, None, 4096,
                         prec=_HI)
    F = _bn_layer_call(Z12, mv(Z12), Ph2, None, None, None, 4096)
    return (F.reshape(B, N, CGEO), gout)
